# Initial kernel scaffold; baseline (speedup 1.0000x reference)
import jax
import jax.numpy as jnp
from jax.experimental import pallas as pl


def kernel(x, edge_index, batch, selected_tasks, W, att_src, att_dst, gat_bias, gn_weight, gn_bias, gn_mean_scale, attn_w1, attn_b1, attn_w2, attn_b2, hid_W, hid_b, fin_W, fin_b, weight_matrix):
    raise NotImplementedError("write your pallas kernel here")


# speedup vs baseline: 31.9857x; 31.9857x over previous
import functools

import jax
import jax.numpy as jnp
from jax import lax
from jax.experimental import pallas as pl
from jax.experimental.pallas import tpu as pltpu
from jax.experimental.pallas import tpu_sc as plsc

_N = 10000
_NP = 10112
_E = 320000
_EP = 327680
_D = 128
_H = 4
_G = 64
_CPT = 160
_CH = 128
_BPT = 20
_RPT = 632
_BLK = 2528
_NBLK = 4
_COMBOS = ((0, 1, 2), (0, 1, 3), (0, 2, 3), (1, 2, 3))


def _logits_body(x_ref, w_ref, as_ref, ad_ref, asn_ref, adn_ref, m_ref,
                 macc_ref):
    i = pl.program_id(0)
    xb = x_ref[...]
    wa_s = []
    wa_d = []
    for h in range(_H):
        wh = w_ref[:, h * _D:(h + 1) * _D]
        wa_s.append(jnp.dot(wh, as_ref[h][:, None],
                            preferred_element_type=jnp.float32))
        wa_d.append(jnp.dot(wh, ad_ref[h][:, None],
                            preferred_element_type=jnp.float32))
    wa = jnp.concatenate(wa_s + wa_d, axis=1)
    a_sd = jnp.dot(xb, wa, preferred_element_type=jnp.float32)
    a_s = a_sd[:, 0:4]
    a_d = a_sd[:, 4:8]
    asn_ref[...] = a_s
    adn_ref[...] = a_d

    bmax = jnp.max(a_s, axis=0)
    cmax = jnp.max(a_d, axis=0)
    upd = jnp.concatenate([jnp.broadcast_to(bmax[:, None], (4, 16)),
                           jnp.broadcast_to(cmax[:, None], (4, 16))], axis=0)

    @pl.when(i == 0)
    def _():
        macc_ref[...] = jnp.full((8, 16), -1e30, jnp.float32)

    macc_ref[...] = jnp.maximum(macc_ref[...], upd)
    acc = macc_ref[...]
    msum = acc[0:4] + acc[4:8]
    ml = jnp.maximum(msum, 0.2 * msum)
    m_ref[...] = jnp.concatenate([ml, jnp.zeros((4, 16), jnp.float32)], axis=0)


def _logits(x_pad, w, as2, ad2):
    return pl.pallas_call(
        _logits_body,
        grid=(_NBLK,),
        in_specs=[
            pl.BlockSpec((_BLK, _D), lambda i: (i, 0)),
            pl.BlockSpec((_D, _H * _D), lambda i: (0, 0)),
            pl.BlockSpec((_H, _D), lambda i: (0, 0)),
            pl.BlockSpec((_H, _D), lambda i: (0, 0)),
        ],
        out_specs=[
            pl.BlockSpec((_BLK, _H), lambda i: (i, 0)),
            pl.BlockSpec((_BLK, _H), lambda i: (i, 0)),
            pl.BlockSpec((8, 16), lambda i: (0, 0)),
        ],
        out_shape=[
            jax.ShapeDtypeStruct((_NP, _H), jnp.float32),
            jax.ShapeDtypeStruct((_NP, _H), jnp.float32),
            jax.ShapeDtypeStruct((8, 16), jnp.float32),
        ],
        scratch_shapes=[pltpu.VMEM((8, 16), jnp.float32)],
        interpret=False,
    )(x_pad, w, as2, ad2)


def _pre_body(x_ref, w_ref, xp_ref):
    xb = x_ref[...]
    xp = jnp.dot(xb, w_ref[...], preferred_element_type=jnp.float32)
    for h in range(_H):
        xp_ref[h, :, :] = xp[:, h * _D:(h + 1) * _D]


def _pre(x_pad, w):
    return pl.pallas_call(
        _pre_body,
        grid=(_NBLK,),
        in_specs=[
            pl.BlockSpec((_BLK, _D), lambda i: (i, 0)),
            pl.BlockSpec((_D, _H * _D), lambda i: (0, 0)),
        ],
        out_specs=[
            pl.BlockSpec((_H, _BLK, _D), lambda i: (0, i, 0)),
        ],
        out_shape=[
            jax.ShapeDtypeStruct((_H, _NP, _D), jnp.float32),
        ],
        interpret=False,
    )(x_pad, w)[0]


@functools.cache
def _alpha_build():
  mesh = plsc.VectorSubcoreMesh(core_axis_name="c", subcore_axis_name="s")

  @functools.partial(
    pl.kernel,
    out_type=[
        jax.ShapeDtypeStruct((_H, 16, _CPT, _CH), jnp.int32),
        jax.ShapeDtypeStruct((_H, 16, _CPT, _CH), jnp.float32),
        jax.ShapeDtypeStruct((_H, _CH, _CH), jnp.float32),
    ],
    mesh=mesh,
    scratch_types=[
        pltpu.VMEM_SHARED((_CH, _CH), jnp.float32),
        pltpu.VMEM((_NP,), jnp.float32),
        pltpu.VMEM((_NP,), jnp.float32),
        pltpu.VMEM((16,), jnp.float32),
        pltpu.VMEM((_CPT, _CH), jnp.int32),
        pltpu.VMEM((_CPT, _CH), jnp.int32),
        pltpu.VMEM((_CPT, _CH), jnp.int32),
        pltpu.VMEM((_CPT, _CH), jnp.float32),
        pltpu.VMEM((_CH, _CH), jnp.float32),
        pltpu.VMEM((1, _CH), jnp.int32),
    ],
    compiler_params=pltpu.CompilerParams(needs_layout_passes=False),
  )
  def _alpha(srcg_hbm, dstg_hbm, bt_hbm, ct_hbm, mb_hbm,
             idx_hbm, al_hbm, asum_hbm,
             asum_sp, b_v, c_v, mb_v, src_v, dst_v, io_v, ao_v, asum_v,
             iden_v):
    c = lax.axis_index("c")
    s = lax.axis_index("s")
    zv = jnp.zeros((16,), jnp.float32)

    pltpu.sync_copy(srcg_hbm.at[s], src_v)
    pltpu.sync_copy(dstg_hbm.at[s], dst_v)
    for v in range(8):
        iden_v[0, pl.ds(v * 16, 16)] = (
            lax.iota(jnp.int32, 16) + jnp.int32(v * 16))

    for hl in range(2):
        head = c * 2 + hl
        pltpu.sync_copy(bt_hbm.at[head], b_v)
        pltpu.sync_copy(ct_hbm.at[head], c_v)
        pltpu.sync_copy(mb_hbm.at[head], mb_v)
        mvec = mb_v[...]
        off = head * _NP

        @pl.loop(0, _CH)
        def _zeroasum(r):
            for q in range(8):
                asum_v[r, pl.ds(q * 16, 16)] = zv

        pltpu.sync_copy(asum_v.at[pl.ds(0, 8)],
                        asum_sp.at[pl.ds(s * 8, 8)])
        plsc.subcore_barrier()

        @pl.loop(0, _CPT)
        def _chunk(j):
            for q in range(8):
                sv = src_v[j, pl.ds(q * 16, 16)]
                dv = dst_v[j, pl.ds(q * 16, 16)]
                t_ = plsc.load_gather(b_v, [sv]) + plsc.load_gather(c_v, [dv])
                alpha = jnp.exp(jnp.maximum(t_, 0.2 * t_) - mvec)
                ao_v[j, pl.ds(q * 16, 16)] = alpha
                io_v[j, pl.ds(q * 16, 16)] = sv + off
                plsc.addupdate_scatter(
                    asum_v,
                    [lax.shift_right_logical(dv, 7),
                     lax.bitwise_and(dv, jnp.int32(127))],
                    alpha)

        pltpu.sync_copy(io_v, idx_hbm.at[head, s])
        pltpu.sync_copy(ao_v, al_hbm.at[head, s])
        pltpu.sync_copy(asum_v, asum_sp.at[iden_v.at[0]], add=True)
        plsc.subcore_barrier()
        pltpu.sync_copy(asum_sp.at[pl.ds(s * 8, 8)],
                        asum_hbm.at[head, pl.ds(s * 8, 8)])
        plsc.subcore_barrier()

  return _alpha


_CH2 = 64
_BPT2 = 40


@functools.cache
def _edge_build():
  mesh = plsc.VectorSubcoreMesh(core_axis_name="c", subcore_axis_name="s")

  @functools.partial(
    pl.kernel,
    out_type=jax.ShapeDtypeStruct((_H, _NP, _D), jnp.float32),
    mesh=mesh,
    scratch_types=[
        pltpu.VMEM_SHARED((_NP, _D), jnp.float32),
        pltpu.VMEM((2, 8, _CH2), jnp.int32),
        pltpu.VMEM((2, 8, _CH2), jnp.int32),
        pltpu.VMEM((2, 8, _CH2), jnp.float32),
        pltpu.VMEM((4, _CH2, _D), jnp.float32),
        pltpu.SemaphoreType.DMA,
        pltpu.SemaphoreType.DMA,
        pltpu.SemaphoreType.DMA,
        pltpu.SemaphoreType.DMA,
        pltpu.SemaphoreType.DMA,
        pltpu.SemaphoreType.DMA,
        pltpu.SemaphoreType.DMA,
        pltpu.SemaphoreType.DMA,
        pltpu.SemaphoreType.DMA,
        pltpu.SemaphoreType.DMA,
        pltpu.SemaphoreType.DMA,
    ],
    compiler_params=pltpu.CompilerParams(needs_layout_passes=False),
  )
  def _edge(xp_hbm, idx_hbm, dstg_hbm, al_hbm, acc_hbm,
            acc_sp, idxb_v, dstb_v, alb_v, rows_v,
            g0, g1, g2, g3, s0, s1, s2, s3, b0, b1, b2):
    c = lax.axis_index("c")
    s = lax.axis_index("s")
    gsems = (g0, g1, g2, g3)
    ssems = (s0, s1, s2, s3)
    base = s * _RPT
    zv = jnp.zeros((16,), jnp.float32)

    def start_gather(pbuf, kk, buf):
        pltpu.async_copy(xp_hbm.at[idxb_v.at[pbuf, kk]], rows_v.at[buf],
                         gsems[buf])

    def wait_gather(pbuf, kk, buf):
        pltpu.make_async_copy(xp_hbm.at[idxb_v.at[pbuf, kk]], rows_v.at[buf],
                              gsems[buf]).wait()

    def start_scatter(pbuf, kk, buf):
        pltpu.async_copy(rows_v.at[buf], acc_sp.at[dstb_v.at[pbuf, kk]],
                         ssems[buf], add=True)

    def wait_scatter(pbuf, kk, buf):
        pltpu.make_async_copy(rows_v.at[buf], acc_sp.at[dstb_v.at[pbuf, kk]],
                              ssems[buf]).wait()

    def start_block(head, pbuf, bk):
        pltpu.async_copy(idx_hbm.at[head, s, pl.ds(bk * 8, 8)],
                         idxb_v.at[pbuf], b0)
        pltpu.async_copy(dstg_hbm.at[s, pl.ds(bk * 8, 8)],
                         dstb_v.at[pbuf], b1)
        pltpu.async_copy(al_hbm.at[head, s, pl.ds(bk * 8, 8)],
                         alb_v.at[pbuf], b2)

    def wait_block(head, pbuf, bk):
        pltpu.make_async_copy(idx_hbm.at[head, s, pl.ds(bk * 8, 8)],
                              idxb_v.at[pbuf], b0).wait()
        pltpu.make_async_copy(dstg_hbm.at[s, pl.ds(bk * 8, 8)],
                              dstb_v.at[pbuf], b1).wait()
        pltpu.make_async_copy(al_hbm.at[head, s, pl.ds(bk * 8, 8)],
                              alb_v.at[pbuf], b2).wait()

    def scale(pbuf, kk, buf):
        alref = alb_v.at[pbuf, kk]

        @plsc.parallel_loop(0, _CH2, unroll=4)
        def _s(r):
            asp = plsc.load_gather(alref, [jnp.zeros((16,), jnp.int32) + r])
            for q in range(8):
                rows_v[buf, r, pl.ds(q * 16, 16)] = (
                    rows_v[buf, r, pl.ds(q * 16, 16)] * asp)

    for hl in range(2):
        head = c * 2 + hl

        @pl.loop(0, _CH2)
        def _zero(r):
            for q in range(8):
                rows_v[0, r, pl.ds(q * 16, 16)] = zv

        for k9 in range(9):
            pltpu.sync_copy(rows_v.at[0],
                            acc_sp.at[pl.ds(base + k9 * 64, 64)])
        pltpu.sync_copy(rows_v.at[0, pl.ds(0, 56)],
                        acc_sp.at[pl.ds(base + 576, 56)])
        plsc.subcore_barrier()

        pltpu.sync_copy(idx_hbm.at[head, s, pl.ds(0, 8)], idxb_v.at[0])
        pltpu.sync_copy(dstg_hbm.at[s, pl.ds(0, 8)], dstb_v.at[0])
        pltpu.sync_copy(al_hbm.at[head, s, pl.ds(0, 8)], alb_v.at[0])
        start_gather(0, 0, 0)
        start_gather(0, 1, 1)

        @pl.loop(0, _BPT2)
        def _blocks(bk):
            pb = lax.rem(bk, 2)
            qb = 1 - pb
            for kk in range(8):
                buf = kk % 4
                buf2 = (kk + 2) % 4
                wait_gather(pb, kk, buf)
                scale(pb, kk, buf)
                start_scatter(pb, kk, buf)
                if kk >= 2:
                    wait_scatter(pb, kk - 2, buf2)
                else:
                    @pl.when(bk > 0)
                    def _ws(kk=kk, buf2=buf2):
                        wait_scatter(qb, 6 + kk, buf2)
                if kk < 6:
                    start_gather(pb, kk + 2, buf2)
                else:
                    @pl.when(bk < _BPT2 - 1)
                    def _sg(kk=kk, buf2=buf2):
                        if kk == 6:
                            wait_block(head, qb, bk + 1)
                        start_gather(qb, kk - 6, buf2)
                if kk == 1:
                    @pl.when(bk < _BPT2 - 1)
                    def _sb():
                        start_block(head, qb, bk + 1)

        wait_scatter(1, 6, 2)
        wait_scatter(1, 7, 3)
        plsc.subcore_barrier()
        pltpu.sync_copy(acc_sp.at[pl.ds(base, _RPT)],
                        acc_hbm.at[head, pl.ds(base, _RPT)])
        plsc.subcore_barrier()

  return _edge


def _tc2a_body(acc_ref, asum_ref, asn_ref, adn_ref, m_ref, xp_ref, gb_ref,
               b3_ref, out_ref, ms_ref, cnt_ref, msacc_ref, cntacc_ref):
    i = pl.program_id(0)
    a_s = asn_ref[...]
    a_d = adn_ref[...]
    asum = asum_ref[...]
    outs = []
    for h in range(_H):
        s_h = a_s[:, h:h + 1] + a_d[:, h:h + 1]
        mh = m_ref[h, 0]
        als = jnp.exp(jnp.maximum(s_h, 0.2 * s_h) - mh)
        num = acc_ref[h, :, :] + als * xp_ref[h, :, :]
        den = asum[:, h:h + 1] + als + 1e-16
        outs.append(num / den)
    out_b = jnp.concatenate(outs, axis=1) + gb_ref[...]
    out_ref[...] = out_b
    bb = b3_ref[0, 0, :]
    oh = (lax.broadcasted_iota(jnp.int32, (_G, _BLK), 0)
          == bb[None, :]).astype(jnp.float32)

    @pl.when(i == 0)
    def _():
        msacc_ref[...] = jnp.zeros((_G, _H * _D), jnp.float32)
        cntacc_ref[...] = jnp.zeros((_G, 1), jnp.float32)

    msacc_ref[...] += jnp.dot(oh, out_b, preferred_element_type=jnp.float32)
    cntacc_ref[...] += jnp.sum(oh, axis=1, keepdims=True)
    ms_ref[...] = msacc_ref[...]
    cnt_ref[...] = cntacc_ref[...]


def _tc2a(acc, asumn, asn, adn, m, xp_aug, gb, b3):
    return pl.pallas_call(
        _tc2a_body,
        grid=(_NBLK,),
        in_specs=[
            pl.BlockSpec((_H, _BLK, _D), lambda i: (0, i, 0)),
            pl.BlockSpec((_BLK, _H), lambda i: (i, 0)),
            pl.BlockSpec((_BLK, _H), lambda i: (i, 0)),
            pl.BlockSpec((_BLK, _H), lambda i: (i, 0)),
            pl.BlockSpec((8, 16), lambda i: (0, 0)),
            pl.BlockSpec((_H, _BLK, _D), lambda i: (0, i, 0)),
            pl.BlockSpec((1, _H * _D), lambda i: (0, 0)),
            pl.BlockSpec((1, 1, _BLK), lambda i: (i, 0, 0)),
        ],
        out_specs=[
            pl.BlockSpec((_BLK, _H * _D), lambda i: (i, 0)),
            pl.BlockSpec((_G, _H * _D), lambda i: (0, 0)),
            pl.BlockSpec((_G, 1), lambda i: (0, 0)),
        ],
        out_shape=[
            jax.ShapeDtypeStruct((_NP, _H * _D), jnp.float32),
            jax.ShapeDtypeStruct((_G, _H * _D), jnp.float32),
            jax.ShapeDtypeStruct((_G, 1), jnp.float32),
        ],
        scratch_shapes=[pltpu.VMEM((_G, _H * _D), jnp.float32),
                        pltpu.VMEM((_G, 1), jnp.float32)],
        interpret=False,
    )(acc, asumn, asn, adn, m, xp_aug, gb, b3)


def _tc2b_body(out_ref, b3_ref, ms_ref, cnt_ref, gms_ref, vs_ref, vsacc_ref):
    i = pl.program_id(0)
    cnt = jnp.maximum(cnt_ref[...], 1.0)
    mean = ms_ref[...] / cnt
    bb = b3_ref[0, 0, :]
    oht = (bb[:, None]
           == lax.broadcasted_iota(jnp.int32, (_BLK, _G), 1)).astype(
               jnp.float32)
    mb = jnp.dot(oht, mean, preferred_element_type=jnp.float32)
    centered = out_ref[...] - gms_ref[...] * mb
    oh = (lax.broadcasted_iota(jnp.int32, (_G, _BLK), 0)
          == bb[None, :]).astype(jnp.float32)

    @pl.when(i == 0)
    def _():
        vsacc_ref[...] = jnp.zeros((_G, _H * _D), jnp.float32)

    vsacc_ref[...] += jnp.dot(oh, centered * centered,
                              preferred_element_type=jnp.float32)
    vs_ref[...] = vsacc_ref[...]


def _tc2b(out_full, b3, ms, cnt, gms):
    return pl.pallas_call(
        _tc2b_body,
        grid=(_NBLK,),
        in_specs=[
            pl.BlockSpec((_BLK, _H * _D), lambda i: (i, 0)),
            pl.BlockSpec((1, 1, _BLK), lambda i: (i, 0, 0)),
            pl.BlockSpec((_G, _H * _D), lambda i: (0, 0)),
            pl.BlockSpec((_G, 1), lambda i: (0, 0)),
            pl.BlockSpec((1, _H * _D), lambda i: (0, 0)),
        ],
        out_specs=[pl.BlockSpec((_G, _H * _D), lambda i: (0, 0))],
        out_shape=[jax.ShapeDtypeStruct((_G, _H * _D), jnp.float32)],
        scratch_shapes=[pltpu.VMEM((_G, _H * _D), jnp.float32)],
        interpret=False,
    )(out_full, b3, ms, cnt, gms)[0]


def _tc2c_body(out_ref, b3_ref, ms_ref, cnt_ref, vs_ref, gms_ref, gnw_ref,
               gnb_ref, w1_ref, b1_ref, w2_ref, b2_ref,
               pn_ref, gs_ref, pnacc_ref, gsacc_ref):
    i = pl.program_id(0)
    cnt = jnp.maximum(cnt_ref[...], 1.0)
    mean = ms_ref[...] / cnt
    var = vs_ref[...] / cnt
    bb = b3_ref[0, 0, :]
    oht = (bb[:, None]
           == lax.broadcasted_iota(jnp.int32, (_BLK, _G), 1)).astype(
               jnp.float32)
    mb = jnp.dot(oht, mean, preferred_element_type=jnp.float32)
    vb = jnp.dot(oht, var, preferred_element_type=jnp.float32)
    centered = out_ref[...] - gms_ref[...] * mb
    xn = centered / jnp.sqrt(vb + 1e-5)
    xg = jnp.maximum(gnw_ref[...] * xn + gnb_ref[...], 0.0)
    h1 = jnp.maximum(
        jnp.dot(xg, w1_ref[...], preferred_element_type=jnp.float32)
        + b1_ref[...], 0.0)
    gl = jnp.dot(h1, w2_ref[...], preferred_element_type=jnp.float32) \
        + b2_ref[...]
    ge = jnp.exp(jax.nn.sigmoid(gl) - 1.0)
    oh = (lax.broadcasted_iota(jnp.int32, (_G, _BLK), 0)
          == bb[None, :]).astype(jnp.float32)

    @pl.when(i == 0)
    def _():
        pnacc_ref[...] = jnp.zeros((_G, _H * _D), jnp.float32)
        gsacc_ref[...] = jnp.zeros((_G, 1), jnp.float32)

    pnacc_ref[...] += jnp.dot(oh, ge * xg, preferred_element_type=jnp.float32)
    gsacc_ref[...] += jnp.dot(oh, ge, preferred_element_type=jnp.float32)
    pn_ref[...] = pnacc_ref[...]
    gs_ref[...] = gsacc_ref[...]


def _tc2c(out_full, b3, ms, cnt, vs, gms, gnw, gnb, w1, b1, w2, b2):
    return pl.pallas_call(
        _tc2c_body,
        grid=(_NBLK,),
        in_specs=[
            pl.BlockSpec((_BLK, _H * _D), lambda i: (i, 0)),
            pl.BlockSpec((1, 1, _BLK), lambda i: (i, 0, 0)),
            pl.BlockSpec((_G, _H * _D), lambda i: (0, 0)),
            pl.BlockSpec((_G, 1), lambda i: (0, 0)),
            pl.BlockSpec((_G, _H * _D), lambda i: (0, 0)),
            pl.BlockSpec((1, _H * _D), lambda i: (0, 0)),
            pl.BlockSpec((1, _H * _D), lambda i: (0, 0)),
            pl.BlockSpec((1, _H * _D), lambda i: (0, 0)),
            pl.BlockSpec((_H * _D, 16), lambda i: (0, 0)),
            pl.BlockSpec((1, 16), lambda i: (0, 0)),
            pl.BlockSpec((16, 1), lambda i: (0, 0)),
            pl.BlockSpec((1, 1), lambda i: (0, 0)),
        ],
        out_specs=[
            pl.BlockSpec((_G, _H * _D), lambda i: (0, 0)),
            pl.BlockSpec((_G, 1), lambda i: (0, 0)),
        ],
        out_shape=[
            jax.ShapeDtypeStruct((_G, _H * _D), jnp.float32),
            jax.ShapeDtypeStruct((_G, 1), jnp.float32),
        ],
        scratch_shapes=[pltpu.VMEM((_G, _H * _D), jnp.float32),
                        pltpu.VMEM((_G, 1), jnp.float32)],
        interpret=False,
    )(out_full, b3, ms, cnt, vs, gms, gnw, gnb, w1, b1, w2, b2)


def _tc2d_body(pn_ref, gs_ref, hw_ref, hb_ref, fw_ref, fb_ref, wm_ref,
               pooled_ref, fins_ref):
    pooled = pn_ref[...] / (gs_ref[...] + 1e-16)
    pooled_ref[...] = pooled
    os_ = []
    for ci in range(4):
        o = jnp.maximum(
            jnp.dot(pooled, hw_ref[ci], preferred_element_type=jnp.float32)
            + hb_ref[ci][None, :], 0.0) * wm_ref[0, ci]
        os_.append(o)
    cols = []
    for t in range(4):
        mem = [os_[ci] for ci, cb in enumerate(_COMBOS) if t in cb]
        comb = (mem[0] + mem[1] + mem[2]) / 3.0
        f = jnp.dot(comb, fw_ref[t], preferred_element_type=jnp.float32) \
            + fb_ref[t, 0]
        cols.append(jax.nn.sigmoid(f))
    fins_ref[...] = jnp.concatenate(cols, axis=1)


def _tc2d(pn, gs, hw, hb, fw, fb, wm):
    return pl.pallas_call(
        _tc2d_body,
        out_shape=[
            jax.ShapeDtypeStruct((_G, _H * _D), jnp.float32),
            jax.ShapeDtypeStruct((_G, 4), jnp.float32),
        ],
        interpret=False,
    )(pn, gs, hw, hb, fw, fb, wm)


def kernel(x, edge_index, batch, selected_tasks, W, att_src, att_dst,
           gat_bias, gn_weight, gn_bias, gn_mean_scale, attn_w1, attn_b1,
           attn_w2, attn_b2, hid_W, hid_b, fin_W, fin_b, weight_matrix):
    del selected_tasks
    f32 = jnp.float32
    i32 = jnp.int32
    ei = edge_index.astype(i32)
    src = jnp.concatenate([ei[0], jnp.full((_EP - _E,), _N, i32)])
    dst = jnp.concatenate([ei[1], jnp.zeros((_EP - _E,), i32)])
    srcg = src.reshape(16, _CPT, _CH)
    dstg = dst.reshape(16, _CPT, _CH)
    x_pad = jnp.pad(x.astype(f32), ((0, _NP - _N), (0, 0)))
    as2 = att_src.astype(f32).reshape(_H, _D)
    ad2 = att_dst.astype(f32).reshape(_H, _D)

    asn, adn, m = _logits(x_pad, W.astype(f32), as2, ad2)

    neg = jnp.full((_H, _NP - _N), -1e30, f32)
    bt = jnp.concatenate([asn[:_N].T, neg], axis=1)
    ct = jnp.concatenate([adn[:_N].T, neg], axis=1)
    eidx, eal, asum3 = _alpha_build()(srcg, dstg, bt, ct, m)
    xp_aug = _pre(x_pad, W.astype(f32))
    acc = _edge_build()(xp_aug.reshape(_H * _NP, _D),
                        eidx.reshape(_H, 16, 8 * _BPT2, _CH2),
                        dstg.reshape(16, 8 * _BPT2, _CH2),
                        eal.reshape(_H, 16, 8 * _BPT2, _CH2))
    asumn = asum3.reshape(_H, _CH * _CH)[:, :_NP].T

    b3 = jnp.concatenate([batch.astype(i32),
                          jnp.full((_NP - _N,), _G, i32)]).reshape(
                              _NBLK, 1, _BLK)
    gb = gat_bias.astype(f32).reshape(1, _H * _D)
    out_full, ms, cnt = _tc2a(acc, asumn, asn, adn, m, xp_aug, gb, b3)
    gms = gn_mean_scale.astype(f32).reshape(1, _H * _D)
    vs = _tc2b(out_full, b3, ms, cnt, gms)
    pn, gs = _tc2c(out_full, b3, ms, cnt, vs, gms,
                   gn_weight.astype(f32).reshape(1, _H * _D),
                   gn_bias.astype(f32).reshape(1, _H * _D),
                   attn_w1.astype(f32), attn_b1.astype(f32).reshape(1, 16),
                   attn_w2.astype(f32), attn_b2.astype(f32).reshape(1, 1))
    pooled, fins = _tc2d(pn, gs, hid_W.astype(f32), hid_b.astype(f32),
                         fin_W.astype(f32), fin_b.astype(f32),
                         weight_matrix.astype(f32).reshape(1, 4))
    return (pooled, fins[:, 0], fins[:, 1], fins[:, 2], fins[:, 3])
